# baseline (speedup 1.0000x reference)
import functools

import jax
import jax.numpy as jnp
from jax import lax
from jax.experimental import pallas as pl
from jax.experimental.pallas import tpu as pltpu
from jax.experimental.pallas import tpu_sc as plsc

_CONF = 0.015
_B, _C = 16384, 1000
_DUMP = _C
_CP = 1024
_BC = 512


_H = 8


def _tc_stats_body(a_ref, g_ref, tgt_ref, nll_ref):
    bc = a_ref.shape[1]
    nt = _C // _H
    z8 = jnp.zeros((_H, bc), jnp.float32)
    neg = jnp.full((_H, bc), -jnp.inf, jnp.float32)
    zi = jnp.zeros((_H, bc), jnp.int32)
    rows = lax.broadcasted_iota(jnp.int32, (_H, bc), 0)

    def pass_a(k, carry):
        acc_s, acc_m, acc_k = carry
        ak = a_ref[pl.ds(pl.multiple_of(k * _H, _H), _H), :]
        acc_s = acc_s + jnp.exp(ak)
        ch = ak > acc_m
        acc_m = jnp.where(ch, ak, acc_m)
        acc_k = jnp.where(ch, k, acc_k)
        return acc_s, acc_m, acc_k

    acc_s, acc_m, acc_k = lax.fori_loop(0, nt, pass_a, (z8, neg, zi))
    idx8 = acc_k * _H + rows
    m = jnp.max(acc_m, axis=0, keepdims=True)
    s0 = jnp.sum(acc_s, axis=0, keepdims=True)
    sel = jnp.exp(m) / s0 > _CONF
    t = jnp.min(jnp.where(acc_m >= m, idx8, _C), axis=0, keepdims=True)

    def pass_g(k, carry):
        acc_g, acc_e = carry
        gk = g_ref[pl.ds(pl.multiple_of(k * _H, _H), _H), :]
        ohb = (k * _H + rows) == t
        acc_g = acc_g + jnp.where(ohb, gk, 0.0)
        acc_e = acc_e + jnp.exp(gk)
        return acc_g, acc_e

    acc_g, acc_e = lax.fori_loop(0, nt, pass_g, (z8, z8))
    gsel = jnp.sum(acc_g, axis=0, keepdims=True)
    sg = jnp.sum(acc_e, axis=0, keepdims=True)
    nll = jnp.log(sg) - gsel

    tgt_ref[...] = jnp.where(sel, t, _DUMP).reshape(-1)
    nll_ref[...] = nll.reshape(-1)


def _tc_stats(anchor_t, aug_t):
    nb = _B // _BC
    return pl.pallas_call(
        _tc_stats_body,
        grid=(nb,),
        in_specs=[
            pl.BlockSpec((_C, _BC), lambda i: (0, i)),
            pl.BlockSpec((_C, _BC), lambda i: (0, i)),
        ],
        out_specs=[
            pl.BlockSpec((_BC,), lambda i: (i,)),
            pl.BlockSpec((_BC,), lambda i: (i,)),
        ],
        out_shape=[
            jax.ShapeDtypeStruct((_B,), jnp.int32),
            jax.ShapeDtypeStruct((_B,), jnp.float32),
        ],
    )(anchor_t, aug_t)


_NW = 16
_CHUNK = _B // _NW


def _sc_loss(tgt, nll):
    mesh = plsc.VectorSubcoreMesh(core_axis_name="c", subcore_axis_name="s")

    @functools.partial(
        pl.kernel,
        mesh=mesh,
        out_type=jax.ShapeDtypeStruct((16,), jnp.float32),
        compiler_params=pltpu.CompilerParams(
            needs_layout_passes=False, use_tc_tiling_on_sc=False),
        scratch_types=[
            pltpu.VMEM((_CHUNK,), jnp.int32),
            pltpu.VMEM((_CHUNK,), jnp.float32),
            pltpu.VMEM((_CP,), jnp.float32),
            pltpu.VMEM((_CP,), jnp.float32),
            pltpu.VMEM((16, 16), jnp.float32),
            pltpu.VMEM((16, 16), jnp.float32),
            pltpu.VMEM((16,), jnp.float32),
            pltpu.VMEM((16,), jnp.float32),
            pltpu.VMEM((16,), jnp.float32),
            pltpu.VMEM_SHARED((_NW, _CP), jnp.float32),
            pltpu.VMEM_SHARED((_NW, _CP), jnp.float32),
            pltpu.VMEM_SHARED((_NW, 16), jnp.float32),
            pltpu.VMEM_SHARED((_NW, 16), jnp.float32),
        ],
    )
    def body(tgt_hbm, nll_hbm, out_hbm, tgt_v, nll_v, counts_v, s_v,
             bufc, bufs, numv, denv, outv, shc, shs, shnum, shden):
        cid = lax.axis_index("c")
        sid = lax.axis_index("s")
        zeros = jnp.zeros((16,), jnp.float32)
        ones = jnp.ones((16,), jnp.float32)
        lane = lax.iota(jnp.int32, 16)

        @pl.when(cid == 0)
        def _():
            pltpu.sync_copy(tgt_hbm.at[pl.ds(sid * _CHUNK, _CHUNK)], tgt_v)
            pltpu.sync_copy(nll_hbm.at[pl.ds(sid * _CHUNK, _CHUNK)], nll_v)

            def zinit(i, carry):
                counts_v[pl.ds(i * 16, 16)] = zeros
                s_v[pl.ds(i * 16, 16)] = zeros
                return carry

            lax.fori_loop(0, _CP // 16, zinit, 0)

            def scat(i, carry):
                idx = tgt_v[pl.ds(i * 16, 16)]
                val = nll_v[pl.ds(i * 16, 16)]
                plsc.addupdate_scatter(counts_v, [idx], ones)
                plsc.addupdate_scatter(s_v, [idx], val)
                return carry

            lax.fori_loop(0, _CHUNK // 16, scat, 0)
            pltpu.sync_copy(counts_v, shc.at[sid])
            pltpu.sync_copy(s_v, shs.at[sid])

        plsc.subcore_barrier()

        @pl.when(cid == 0)
        def _():
            def chunkloop(j, carry):
                num, den = carry
                ch = sid * 4 + j
                pltpu.sync_copy(shc.at[:, pl.ds(ch * 16, 16)], bufc)
                pltpu.sync_copy(shs.at[:, pl.ds(ch * 16, 16)], bufs)

                def rowsum(r, cc):
                    tc, ts = cc
                    return tc + bufc[r], ts + bufs[r]

                tc, ts = lax.fori_loop(0, 16, rowsum, (zeros, zeros))
                valid = (ch * 16 + lane) < _C
                present = jnp.logical_and(valid, tc > 0.0)
                contrib = jnp.where(present, ts / jnp.maximum(tc, 1.0), 0.0)
                return num + contrib, den + jnp.where(present, ones, zeros)

            num, den = lax.fori_loop(0, 4, chunkloop, (zeros, zeros))
            numv[...] = num
            denv[...] = den
            pltpu.sync_copy(numv, shnum.at[sid])
            pltpu.sync_copy(denv, shden.at[sid])

        plsc.subcore_barrier()

        @pl.when(jnp.logical_and(cid == 0, sid == 0))
        def _():
            pltpu.sync_copy(shnum, bufc)
            pltpu.sync_copy(shden, bufs)

            def rowsum2(r, cc):
                tn, td = cc
                return tn + bufc[r], td + bufs[r]

            tn, td = lax.fori_loop(0, 16, rowsum2, (zeros, zeros))
            num_s = jnp.broadcast_to(jnp.sum(tn), (16,))
            den_s = jnp.broadcast_to(jnp.sum(td), (16,))
            outv[...] = num_s / den_s
            pltpu.sync_copy(outv, out_hbm)

    return body(tgt, nll)


def kernel(anchor_logits, aug_logits):
    tgt, nll = _tc_stats(anchor_logits.T, aug_logits.T)
    out = _sc_loss(tgt, nll)
    return out[0]


# speedup vs baseline: 1.9508x; 1.9508x over previous
import functools

import jax
import jax.numpy as jnp
from jax import lax
from jax.experimental import pallas as pl
from jax.experimental.pallas import tpu as pltpu
from jax.experimental.pallas import tpu_sc as plsc

_CONF = 0.015
_B, _C = 16384, 1000
_DUMP = _C
_CP = 1024
_BC = 2048


def _tc_stats_body(a_ref, g_ref, tgt_ref, nll_ref):
    a = a_ref[...]
    row = lax.broadcasted_iota(jnp.int32, a.shape, 0)
    m = jnp.max(a, axis=0, keepdims=True)
    s0 = jnp.sum(jnp.exp(a), axis=0, keepdims=True)
    sel = jnp.exp(m) / s0 > _CONF
    t = jnp.min(jnp.where(a >= m, row, _C), axis=0, keepdims=True)

    g = g_ref[...]
    sg = jnp.sum(jnp.exp(g), axis=0, keepdims=True)
    gsel = jnp.sum(jnp.where(row == t, g, 0.0), axis=0, keepdims=True)
    nll = jnp.log(sg) - gsel

    tgt_ref[...] = jnp.where(sel, t, _DUMP).reshape(-1)
    nll_ref[...] = nll.reshape(-1)


def _tc_stats(anchor_t, aug_t):
    nb = _B // _BC
    return pl.pallas_call(
        _tc_stats_body,
        grid=(nb,),
        in_specs=[
            pl.BlockSpec((_C, _BC), lambda i: (0, i)),
            pl.BlockSpec((_C, _BC), lambda i: (0, i)),
        ],
        out_specs=[
            pl.BlockSpec((_BC,), lambda i: (i,)),
            pl.BlockSpec((_BC,), lambda i: (i,)),
        ],
        out_shape=[
            jax.ShapeDtypeStruct((_B,), jnp.int32),
            jax.ShapeDtypeStruct((_B,), jnp.float32),
        ],
    )(anchor_t, aug_t)


_NW = 16
_CHUNK = _B // _NW


def _sc_loss(tgt, nll):
    mesh = plsc.VectorSubcoreMesh(core_axis_name="c", subcore_axis_name="s")

    @functools.partial(
        pl.kernel,
        mesh=mesh,
        out_type=jax.ShapeDtypeStruct((16,), jnp.float32),
        compiler_params=pltpu.CompilerParams(
            needs_layout_passes=False, use_tc_tiling_on_sc=False),
        scratch_types=[
            pltpu.VMEM((_CHUNK,), jnp.int32),
            pltpu.VMEM((_CHUNK,), jnp.float32),
            pltpu.VMEM((_CP,), jnp.float32),
            pltpu.VMEM((_CP,), jnp.float32),
            pltpu.VMEM((16, 16), jnp.float32),
            pltpu.VMEM((16, 16), jnp.float32),
            pltpu.VMEM((16,), jnp.float32),
            pltpu.VMEM((16,), jnp.float32),
            pltpu.VMEM((16,), jnp.float32),
            pltpu.VMEM_SHARED((_NW, _CP), jnp.float32),
            pltpu.VMEM_SHARED((_NW, _CP), jnp.float32),
            pltpu.VMEM_SHARED((_NW, 16), jnp.float32),
            pltpu.VMEM_SHARED((_NW, 16), jnp.float32),
        ],
    )
    def body(tgt_hbm, nll_hbm, out_hbm, tgt_v, nll_v, counts_v, s_v,
             bufc, bufs, numv, denv, outv, shc, shs, shnum, shden):
        cid = lax.axis_index("c")
        sid = lax.axis_index("s")
        zeros = jnp.zeros((16,), jnp.float32)
        ones = jnp.ones((16,), jnp.float32)
        lane = lax.iota(jnp.int32, 16)

        @pl.when(cid == 0)
        def _():
            pltpu.sync_copy(tgt_hbm.at[pl.ds(sid * _CHUNK, _CHUNK)], tgt_v)
            pltpu.sync_copy(nll_hbm.at[pl.ds(sid * _CHUNK, _CHUNK)], nll_v)

            def zinit(i, carry):
                counts_v[pl.ds(i * 16, 16)] = zeros
                s_v[pl.ds(i * 16, 16)] = zeros
                return carry

            lax.fori_loop(0, _CP // 16, zinit, 0)

            def scat(i, carry):
                idx = tgt_v[pl.ds(i * 16, 16)]
                val = nll_v[pl.ds(i * 16, 16)]
                plsc.addupdate_scatter(counts_v, [idx], ones)
                plsc.addupdate_scatter(s_v, [idx], val)
                return carry

            lax.fori_loop(0, _CHUNK // 16, scat, 0)
            pltpu.sync_copy(counts_v, shc.at[sid])
            pltpu.sync_copy(s_v, shs.at[sid])

        plsc.subcore_barrier()

        @pl.when(cid == 0)
        def _():
            def chunkloop(j, carry):
                num, den = carry
                ch = sid * 4 + j
                pltpu.sync_copy(shc.at[:, pl.ds(ch * 16, 16)], bufc)
                pltpu.sync_copy(shs.at[:, pl.ds(ch * 16, 16)], bufs)

                def rowsum(r, cc):
                    tc, ts = cc
                    return tc + bufc[r], ts + bufs[r]

                tc, ts = lax.fori_loop(0, 16, rowsum, (zeros, zeros))
                valid = (ch * 16 + lane) < _C
                present = jnp.logical_and(valid, tc > 0.0)
                contrib = jnp.where(present, ts / jnp.maximum(tc, 1.0), 0.0)
                return num + contrib, den + jnp.where(present, ones, zeros)

            num, den = lax.fori_loop(0, 4, chunkloop, (zeros, zeros))
            numv[...] = num
            denv[...] = den
            pltpu.sync_copy(numv, shnum.at[sid])
            pltpu.sync_copy(denv, shden.at[sid])

        plsc.subcore_barrier()

        @pl.when(jnp.logical_and(cid == 0, sid == 0))
        def _():
            pltpu.sync_copy(shnum, bufc)
            pltpu.sync_copy(shden, bufs)

            def rowsum2(r, cc):
                tn, td = cc
                return tn + bufc[r], td + bufs[r]

            tn, td = lax.fori_loop(0, 16, rowsum2, (zeros, zeros))
            num_s = jnp.broadcast_to(jnp.sum(tn), (16,))
            den_s = jnp.broadcast_to(jnp.sum(td), (16,))
            outv[...] = num_s / den_s
            pltpu.sync_copy(outv, out_hbm)

    return body(tgt, nll)


def kernel(anchor_logits, aug_logits):
    tgt, nll = _tc_stats(anchor_logits.T, aug_logits.T)
    out = _sc_loss(tgt, nll)
    return out[0]
